# baseline (speedup 1.0000x reference)
import functools

import jax
import jax.numpy as jnp
from jax import lax
from jax.experimental import pallas as pl
from jax.experimental.pallas import tpu as pltpu
from jax.experimental.pallas import tpu_sc as plsc

N = 10000
E = 320000
D_IN = 128
D_OUT = 7

NC = 2
NS = 16
NW = NC * NS
CPW = D_IN // NW
EBLK = 8000
NEG = -3.0e38


def _sc_segment_max_kernel(xt_hbm, src_hbm, dst_hbm, aggt_hbm,
                           xcol, agg, srcb, dstb):
    wid = lax.axis_index("s") * NC + lax.axis_index("c")
    pltpu.sync_copy(xt_hbm.at[pl.ds(wid * (CPW * N), CPW * N)], xcol)

    ninf = jnp.full((16,), NEG, jnp.float32)

    def init_body(i, _):
        agg[pl.ds(i * 16, 16)] = ninf
        return 0

    lax.fori_loop(0, (CPW * N) // 16, init_body, 0)

    def edge_block(b, _):
        pltpu.sync_copy(src_hbm.at[pl.ds(b * EBLK, EBLK)], srcb)
        pltpu.sync_copy(dst_hbm.at[pl.ds(b * EBLK, EBLK)], dstb)

        def chunk(i, _):
            sv = srcb[pl.ds(i * 16, 16)]
            dv = dstb[pl.ds(i * 16, 16)]
            for k in range(CPW):
                svk = sv + (k * N)
                dvk = dv + (k * N)
                val = plsc.load_gather(xcol, [svk])
                cur = plsc.load_gather(agg, [dvk])
                m = jnp.maximum(val, cur)
                plsc.store_scatter(agg, [dvk], m)
                def fix_cond(chk):
                    return jnp.any(chk < m)

                def fix_body(chk):
                    plsc.store_scatter(agg, [dvk], m, mask=chk < m)
                    return plsc.load_gather(agg, [dvk])

                chk0 = plsc.load_gather(agg, [dvk])
                lax.while_loop(fix_cond, fix_body, chk0)
            return 0

        lax.fori_loop(0, EBLK // 16, chunk, 0)
        return 0

    lax.fori_loop(0, E // EBLK, edge_block, 0)
    pltpu.sync_copy(agg, aggt_hbm.at[pl.ds(wid * (CPW * N), CPW * N)])


@functools.partial(jax.jit, static_argnames=())
def _sc_segment_max(xt_flat, src, dst):
    mesh = plsc.VectorSubcoreMesh(core_axis_name="c", subcore_axis_name="s",
                                  num_cores=NC, num_subcores=NS)
    return pl.kernel(
        _sc_segment_max_kernel,
        out_type=jax.ShapeDtypeStruct((D_IN * N,), jnp.float32),
        mesh=mesh,
        compiler_params=pltpu.CompilerParams(needs_layout_passes=False),
        scratch_types=[
            pltpu.VMEM((CPW * N,), jnp.float32),
            pltpu.VMEM((CPW * N,), jnp.float32),
            pltpu.VMEM((EBLK,), jnp.int32),
            pltpu.VMEM((EBLK,), jnp.int32),
        ],
    )(xt_flat, src, dst)


def _epilogue_kernel(aggt_ref, xt_ref, wl_ref, wr_ref, b_ref, out_ref):
    aggt = aggt_ref[...]
    aggt = jnp.where(aggt <= NEG, 0.0, aggt)
    outt = (
        lax.dot_general(wl_ref[...], aggt, (((0,), (0,)), ((), ())))
        + lax.dot_general(wr_ref[...], xt_ref[...], (((0,), (0,)), ((), ())))
        + b_ref[...]
    )
    s = outt - jnp.max(outt, axis=0, keepdims=True)
    out_ref[...] = s - jnp.log(jnp.sum(jnp.exp(s), axis=0, keepdims=True))


def _epilogue(aggt, xt, W_l, W_r, b):
    return pl.pallas_call(
        _epilogue_kernel,
        out_shape=jax.ShapeDtypeStruct((D_OUT, N), jnp.float32),
    )(aggt, xt, W_l, W_r, b)


def kernel(x, edge_index, W_l, W_r, b):
    src = edge_index[0].astype(jnp.int32)
    dst = edge_index[1].astype(jnp.int32)
    xt = x.T
    aggt = _sc_segment_max(xt.reshape(-1), src, dst).reshape(D_IN, N)
    outt = _epilogue(aggt, xt, W_l, W_r, b.reshape(D_OUT, 1))
    return outt.T


# speedup vs baseline: 2.8995x; 2.8995x over previous
import functools

import jax
import jax.numpy as jnp
from jax import lax
from jax.experimental import pallas as pl
from jax.experimental.pallas import tpu as pltpu
from jax.experimental.pallas import tpu_sc as plsc

N = 10000
E = 320000
D_IN = 128
D_OUT = 7

NC = 2
NS = 16
NW = NC * NS
CPW = D_IN // NW
EBLK = 8000
NEG = -3.0e38


def _sc_segment_max_kernel(xt_hbm, src_hbm, dst_hbm, aggt_hbm,
                           xcol, agg, srcb, dstb, dupb, rsrc, rdst):
    wid = lax.axis_index("s") * NC + lax.axis_index("c")
    pltpu.sync_copy(xt_hbm.at[pl.ds(wid * (CPW * N), CPW * N)], xcol)

    ninf = jnp.full((16,), NEG, jnp.float32)
    lanes = lax.iota(jnp.int32, 16)

    def init_body(i, _):
        agg[pl.ds(i * 16, 16)] = ninf
        return 0

    lax.fori_loop(0, (CPW * N) // 16, init_body, 0)

    def edge_block(b, _):
        pltpu.sync_copy(src_hbm.at[pl.ds(b * EBLK, EBLK)], srcb)
        pltpu.sync_copy(dst_hbm.at[pl.ds(b * EBLK, EBLK)], dstb)

        def chunk(i, offv):
            sv = srcb[pl.ds(i * 16, 16)]
            dv = dstb[pl.ds(i * 16, 16)]
            plsc.store_scatter(dupb, [dv], lanes)
            w1 = plsc.load_gather(dupb, [dv])
            lost1 = w1 != lanes
            plsc.store_scatter(dupb, [dv], lanes, mask=lost1)
            w2 = plsc.load_gather(dupb, [dv])
            conf = lost1 | (w2 != lanes)
            for k in range(CPW):
                val = plsc.load_gather(xcol, [sv + (k * N)])
                cur = plsc.load_gather(agg, [dv + (k * N)])
                plsc.store_scatter(agg, [dv + (k * N)], jnp.maximum(val, cur))
            ranks = plsc.cumsum(conf.astype(jnp.int32))
            pos = offv + ranks - 1
            plsc.store_scatter(rsrc, [pos], sv, mask=conf)
            plsc.store_scatter(rdst, [pos], dv, mask=conf)
            return offv + plsc.all_reduce_population_count(conf)

        offv = lax.fori_loop(0, EBLK // 16, chunk, jnp.zeros((16,), jnp.int32))

        cnt = jnp.max(offv)

        def retry_chunk(j, _):
            valid = (j * 16 + lanes) < offv
            sv = jnp.where(valid, rsrc[pl.ds(j * 16, 16)], 0)
            dv = jnp.where(valid, rdst[pl.ds(j * 16, 16)], 0)
            for k in range(CPW):
                val = plsc.load_gather(xcol, [sv + (k * N)])
                cur = plsc.load_gather(agg, [dv + (k * N)])
                m = jnp.where(valid, jnp.maximum(val, cur), NEG)
                plsc.store_scatter(agg, [dv + (k * N)], m, mask=valid)

                def fix_cond(chk):
                    return jnp.any(chk < m)

                def fix_body(chk):
                    plsc.store_scatter(agg, [dv + (k * N)], m, mask=chk < m)
                    return plsc.load_gather(agg, [dv + (k * N)])

                lax.while_loop(fix_cond, fix_body,
                               plsc.load_gather(agg, [dv + (k * N)]))
            return 0

        lax.fori_loop(0, (cnt + 15) // 16, retry_chunk, 0)
        return 0

    lax.fori_loop(0, E // EBLK, edge_block, 0)
    pltpu.sync_copy(agg, aggt_hbm.at[pl.ds(wid * (CPW * N), CPW * N)])


@functools.partial(jax.jit, static_argnames=())
def _sc_segment_max(xt_flat, src, dst):
    mesh = plsc.VectorSubcoreMesh(core_axis_name="c", subcore_axis_name="s",
                                  num_cores=NC, num_subcores=NS)
    return pl.kernel(
        _sc_segment_max_kernel,
        out_type=jax.ShapeDtypeStruct((D_IN * N,), jnp.float32),
        mesh=mesh,
        compiler_params=pltpu.CompilerParams(needs_layout_passes=False),
        scratch_types=[
            pltpu.VMEM((CPW * N,), jnp.float32),
            pltpu.VMEM((CPW * N,), jnp.float32),
            pltpu.VMEM((EBLK,), jnp.int32),
            pltpu.VMEM((EBLK,), jnp.int32),
            pltpu.VMEM((N,), jnp.int32),
            pltpu.VMEM((EBLK,), jnp.int32),
            pltpu.VMEM((EBLK,), jnp.int32),
        ],
    )(xt_flat, src, dst)


def _epilogue_kernel(aggt_ref, xt_ref, wl_ref, wr_ref, b_ref, out_ref):
    aggt = aggt_ref[...]
    aggt = jnp.where(aggt <= NEG, 0.0, aggt)
    outt = (
        lax.dot_general(wl_ref[...], aggt, (((0,), (0,)), ((), ())))
        + lax.dot_general(wr_ref[...], xt_ref[...], (((0,), (0,)), ((), ())))
        + b_ref[...]
    )
    s = outt - jnp.max(outt, axis=0, keepdims=True)
    out_ref[...] = s - jnp.log(jnp.sum(jnp.exp(s), axis=0, keepdims=True))


def _epilogue(aggt, xt, W_l, W_r, b):
    return pl.pallas_call(
        _epilogue_kernel,
        out_shape=jax.ShapeDtypeStruct((D_OUT, N), jnp.float32),
    )(aggt, xt, W_l, W_r, b)


def kernel(x, edge_index, W_l, W_r, b):
    src = edge_index[0].astype(jnp.int32)
    dst = edge_index[1].astype(jnp.int32)
    xt = x.T
    aggt = _sc_segment_max(xt.reshape(-1), src, dst).reshape(D_IN, N)
    outt = _epilogue(aggt, xt, W_l, W_r, b.reshape(D_OUT, 1))
    return outt.T


# speedup vs baseline: 3.6657x; 1.2642x over previous
import functools

import jax
import jax.numpy as jnp
from jax import lax
from jax.experimental import pallas as pl
from jax.experimental.pallas import tpu as pltpu
from jax.experimental.pallas import tpu_sc as plsc

N = 10000
E = 320000
D_IN = 128
D_OUT = 7

NC = 2
NS = 16
NW = NC * NS
CPW = D_IN // NW
EBLK = 8000
NEG = -3.0e38


def _sc_segment_max_kernel(xt_hbm, src_hbm, dst_hbm, aggt_hbm,
                           xcol, agg, srcb, dstb, rsrc, rdst):
    wid = lax.axis_index("s") * NC + lax.axis_index("c")
    pltpu.sync_copy(xt_hbm.at[pl.ds(wid * (CPW * N), CPW * N)], xcol)

    ninf = jnp.full((16,), NEG, jnp.float32)
    lanes = lax.iota(jnp.int32, 16)

    def init_body(i, _):
        agg[pl.ds(i * 16, 16)] = ninf
        return 0

    lax.fori_loop(0, (CPW * N) // 16, init_body, 0)

    def edge_block(b, _):
        pltpu.sync_copy(src_hbm.at[pl.ds(b * EBLK, EBLK)], srcb)
        pltpu.sync_copy(dst_hbm.at[pl.ds(b * EBLK, EBLK)], dstb)

        def chunk(i, offv):
            for u in range(2):
                sv = srcb[pl.ds((2 * i + u) * 16, 16)]
                dv = dstb[pl.ds((2 * i + u) * 16, 16)]
                cnt, last = plsc.scan_count(dv)
                conf = (cnt != 1) | jnp.logical_not(last)
                for k in range(CPW):
                    val = plsc.load_gather(xcol, [sv + (k * N)])
                    cur = plsc.load_gather(agg, [dv + (k * N)])
                    plsc.store_scatter(agg, [dv + (k * N)],
                                       jnp.maximum(val, cur))
                ranks = plsc.cumsum(conf.astype(jnp.int32))
                pos = offv + ranks - 1
                plsc.store_scatter(rsrc, [pos], sv, mask=conf)
                plsc.store_scatter(rdst, [pos], dv, mask=conf)
                offv = offv + plsc.all_reduce_population_count(conf)
            return offv

        offv = lax.fori_loop(0, EBLK // 32, chunk, jnp.zeros((16,), jnp.int32))

        cnt = jnp.max(offv)

        def retry_chunk(j, _):
            valid = (j * 16 + lanes) < offv
            sv = jnp.where(valid, rsrc[pl.ds(j * 16, 16)], 0)
            dv = jnp.where(valid, rdst[pl.ds(j * 16, 16)], 0)
            for k in range(CPW):
                val = plsc.load_gather(xcol, [sv + (k * N)])
                cur = plsc.load_gather(agg, [dv + (k * N)])
                m = jnp.where(valid, jnp.maximum(val, cur), NEG)
                plsc.store_scatter(agg, [dv + (k * N)], m, mask=valid)

                def fix_cond(chk):
                    return jnp.any(chk < m)

                def fix_body(chk):
                    plsc.store_scatter(agg, [dv + (k * N)], m, mask=chk < m)
                    return plsc.load_gather(agg, [dv + (k * N)])

                lax.while_loop(fix_cond, fix_body,
                               plsc.load_gather(agg, [dv + (k * N)]))
            return 0

        lax.fori_loop(0, (cnt + 15) // 16, retry_chunk, 0)
        return 0

    lax.fori_loop(0, E // EBLK, edge_block, 0)
    pltpu.sync_copy(agg, aggt_hbm.at[pl.ds(wid * (CPW * N), CPW * N)])


@functools.partial(jax.jit, static_argnames=())
def _sc_segment_max(xt_flat, src, dst):
    mesh = plsc.VectorSubcoreMesh(core_axis_name="c", subcore_axis_name="s",
                                  num_cores=NC, num_subcores=NS)
    return pl.kernel(
        _sc_segment_max_kernel,
        out_type=jax.ShapeDtypeStruct((D_IN * N,), jnp.float32),
        mesh=mesh,
        compiler_params=pltpu.CompilerParams(needs_layout_passes=False),
        scratch_types=[
            pltpu.VMEM((CPW * N,), jnp.float32),
            pltpu.VMEM((CPW * N,), jnp.float32),
            pltpu.VMEM((EBLK,), jnp.int32),
            pltpu.VMEM((EBLK,), jnp.int32),
            pltpu.VMEM((EBLK,), jnp.int32),
            pltpu.VMEM((EBLK,), jnp.int32),
        ],
    )(xt_flat, src, dst)


def _epilogue_kernel(aggt_ref, xt_ref, wl_ref, wr_ref, b_ref, out_ref):
    aggt = aggt_ref[...]
    aggt = jnp.where(aggt <= NEG, 0.0, aggt)
    outt = (
        lax.dot_general(wl_ref[...], aggt, (((0,), (0,)), ((), ())))
        + lax.dot_general(wr_ref[...], xt_ref[...], (((0,), (0,)), ((), ())))
        + b_ref[...]
    )
    s = outt - jnp.max(outt, axis=0, keepdims=True)
    out_ref[...] = s - jnp.log(jnp.sum(jnp.exp(s), axis=0, keepdims=True))


def _epilogue(aggt, xt, W_l, W_r, b):
    return pl.pallas_call(
        _epilogue_kernel,
        out_shape=jax.ShapeDtypeStruct((D_OUT, N), jnp.float32),
    )(aggt, xt, W_l, W_r, b)


def kernel(x, edge_index, W_l, W_r, b):
    src = edge_index[0].astype(jnp.int32)
    dst = edge_index[1].astype(jnp.int32)
    xt = x.T
    aggt = _sc_segment_max(xt.reshape(-1), src, dst).reshape(D_IN, N)
    outt = _epilogue(aggt, xt, W_l, W_r, b.reshape(D_OUT, 1))
    return outt.T


# speedup vs baseline: 5.1401x; 1.4022x over previous
import functools

import jax
import jax.numpy as jnp
from jax import lax
from jax.experimental import pallas as pl
from jax.experimental.pallas import tpu as pltpu
from jax.experimental.pallas import tpu_sc as plsc

N = 10000
E = 320000
D_IN = 128
D_OUT = 7

NC = 2
NS = 16
NW = NC * NS
CPW = D_IN // NW
EBLK = 8000
NEG = -3.0e38


def _sc_segment_max_kernel(xt_hbm, src_hbm, dst_hbm, aggt_hbm, *scratch):
    xcols = scratch[0:CPW]
    aggs = scratch[CPW:2 * CPW]
    srcb, dstb, rsrc, rdst = scratch[2 * CPW:]
    wid = lax.axis_index("s") * NC + lax.axis_index("c")
    for k in range(CPW):
        pltpu.sync_copy(xt_hbm.at[pl.ds((wid * CPW + k) * N, N)], xcols[k])

    ninf = jnp.full((16,), NEG, jnp.float32)
    lanes = lax.iota(jnp.int32, 16)

    def init_body(i, _):
        for k in range(CPW):
            aggs[k][pl.ds(i * 16, 16)] = ninf
        return 0

    lax.fori_loop(0, N // 16, init_body, 0)

    def edge_block(b, _):
        pltpu.sync_copy(src_hbm.at[pl.ds(b * EBLK, EBLK)], srcb)
        pltpu.sync_copy(dst_hbm.at[pl.ds(b * EBLK, EBLK)], dstb)

        def chunk(i, offv):
            svs = [srcb[pl.ds((2 * i + u) * 16, 16)] for u in range(2)]
            dvs = [dstb[pl.ds((2 * i + u) * 16, 16)] for u in range(2)]
            scans = [plsc.scan_count(dvs[u]) for u in range(2)]
            vals = [[plsc.load_gather(xcols[k], [svs[u]])
                     for k in range(CPW)] for u in range(2)]
            for u in range(2):
                sv, dv = svs[u], dvs[u]
                cnt, last = scans[u]
                conf = (cnt != 1) | jnp.logical_not(last)
                curs = [plsc.load_gather(aggs[k], [dv]) for k in range(CPW)]
                for k in range(CPW):
                    plsc.store_scatter(aggs[k], [dv],
                                       jnp.maximum(vals[u][k], curs[k]))
                ranks = plsc.cumsum(conf.astype(jnp.int32))
                pos = offv + ranks - 1
                plsc.store_scatter(rsrc, [pos], sv, mask=conf)
                plsc.store_scatter(rdst, [pos], dv, mask=conf)
                offv = offv + plsc.all_reduce_population_count(conf)
            return offv

        offv = lax.fori_loop(0, EBLK // 32, chunk, jnp.zeros((16,), jnp.int32))

        cnt = jnp.max(offv)

        def retry_chunk(j, _):
            valid = (j * 16 + lanes) < offv
            sv = jnp.where(valid, rsrc[pl.ds(j * 16, 16)], 0)
            dv = jnp.where(valid, rdst[pl.ds(j * 16, 16)], 0)
            for k in range(CPW):
                val = plsc.load_gather(xcols[k], [sv])
                cur = plsc.load_gather(aggs[k], [dv])
                m = jnp.where(valid, jnp.maximum(val, cur), NEG)
                plsc.store_scatter(aggs[k], [dv], m, mask=valid)

                def fix_cond(chk):
                    return jnp.any(chk < m)

                def fix_body(chk):
                    plsc.store_scatter(aggs[k], [dv], m, mask=chk < m)
                    return plsc.load_gather(aggs[k], [dv])

                lax.while_loop(fix_cond, fix_body,
                               plsc.load_gather(aggs[k], [dv]))
            return 0

        lax.fori_loop(0, (cnt + 15) // 16, retry_chunk, 0)
        return 0

    lax.fori_loop(0, E // EBLK, edge_block, 0)
    for k in range(CPW):
        pltpu.sync_copy(aggs[k], aggt_hbm.at[pl.ds((wid * CPW + k) * N, N)])


@functools.partial(jax.jit, static_argnames=())
def _sc_segment_max(xt_flat, src, dst):
    mesh = plsc.VectorSubcoreMesh(core_axis_name="c", subcore_axis_name="s",
                                  num_cores=NC, num_subcores=NS)
    return pl.kernel(
        _sc_segment_max_kernel,
        out_type=jax.ShapeDtypeStruct((D_IN * N,), jnp.float32),
        mesh=mesh,
        compiler_params=pltpu.CompilerParams(needs_layout_passes=False),
        scratch_types=(
            [pltpu.VMEM((N,), jnp.float32) for _ in range(2 * CPW)]
            + [pltpu.VMEM((EBLK,), jnp.int32) for _ in range(4)]
        ),
    )(xt_flat, src, dst)


def _epilogue_kernel(aggt_ref, xt_ref, wl_ref, wr_ref, b_ref, out_ref):
    aggt = aggt_ref[...]
    aggt = jnp.where(aggt <= NEG, 0.0, aggt)
    outt = (
        lax.dot_general(wl_ref[...], aggt, (((0,), (0,)), ((), ())))
        + lax.dot_general(wr_ref[...], xt_ref[...], (((0,), (0,)), ((), ())))
        + b_ref[...]
    )
    s = outt - jnp.max(outt, axis=0, keepdims=True)
    out_ref[...] = s - jnp.log(jnp.sum(jnp.exp(s), axis=0, keepdims=True))


def _epilogue(aggt, xt, W_l, W_r, b):
    return pl.pallas_call(
        _epilogue_kernel,
        out_shape=jax.ShapeDtypeStruct((D_OUT, N), jnp.float32),
    )(aggt, xt, W_l, W_r, b)


def kernel(x, edge_index, W_l, W_r, b):
    src = edge_index[0].astype(jnp.int32)
    dst = edge_index[1].astype(jnp.int32)
    xt = x.T
    aggt = _sc_segment_max(xt.reshape(-1), src, dst).reshape(D_IN, N)
    outt = _epilogue(aggt, xt, W_l, W_r, b.reshape(D_OUT, 1))
    return outt.T


# speedup vs baseline: 5.9889x; 1.1651x over previous
import functools

import jax
import jax.numpy as jnp
from jax import lax
from jax.experimental import pallas as pl
from jax.experimental.pallas import tpu as pltpu
from jax.experimental.pallas import tpu_sc as plsc

N = 10000
E = 320000
D_IN = 128
D_OUT = 7

NC = 2
NS = 16
NW = NC * NS
CPW = D_IN // NW
EBLK = 8000
NEG = -3.0e38


def _sc_segment_max_kernel(xt_hbm, src_hbm, dst_hbm, aggt_hbm, *scratch):
    xcols = scratch[0:CPW]
    aggs = scratch[CPW:2 * CPW]
    srcb0, dstb0, srcb1, dstb1, rsrc, rdst = scratch[2 * CPW:2 * CPW + 6]
    sems = scratch[2 * CPW + 6]
    wid = lax.axis_index("s") * NC + lax.axis_index("c")
    for k in range(CPW):
        pltpu.sync_copy(xt_hbm.at[pl.ds((wid * CPW + k) * N, N)], xcols[k])

    ninf = jnp.full((16,), NEG, jnp.float32)
    lanes = lax.iota(jnp.int32, 16)

    def init_body(i, _):
        for k in range(CPW):
            aggs[k][pl.ds(i * 16, 16)] = ninf
        return 0

    lax.fori_loop(0, N // 16, init_body, 0)

    NB = E // EBLK

    def fetch(b, srcb, dstb, sp):
        pltpu.async_copy(src_hbm.at[pl.ds(b * EBLK, EBLK)], srcb, sems.at[sp])
        pltpu.async_copy(dst_hbm.at[pl.ds(b * EBLK, EBLK)], dstb,
                         sems.at[sp + 1])

    def wait(b, srcb, dstb, sp):
        pltpu.make_async_copy(src_hbm.at[pl.ds(b * EBLK, EBLK)], srcb,
                              sems.at[sp]).wait()
        pltpu.make_async_copy(dst_hbm.at[pl.ds(b * EBLK, EBLK)], dstb,
                              sems.at[sp + 1]).wait()

    def process(srcb, dstb):
        def chunk(i, offv):
            svs = [srcb[pl.ds((2 * i + u) * 16, 16)] for u in range(2)]
            dvs = [dstb[pl.ds((2 * i + u) * 16, 16)] for u in range(2)]
            scans = [plsc.scan_count(dvs[u]) for u in range(2)]
            vals = [[plsc.load_gather(xcols[k], [svs[u]])
                     for k in range(CPW)] for u in range(2)]
            for u in range(2):
                sv, dv = svs[u], dvs[u]
                cnt, last = scans[u]
                conf = (cnt != 1) | jnp.logical_not(last)
                curs = [plsc.load_gather(aggs[k], [dv]) for k in range(CPW)]
                for k in range(CPW):
                    plsc.store_scatter(aggs[k], [dv],
                                       jnp.maximum(vals[u][k], curs[k]))
                ranks = plsc.cumsum(conf.astype(jnp.int32))
                pos = offv + ranks - 1
                plsc.store_scatter(rsrc, [pos], sv, mask=conf)
                plsc.store_scatter(rdst, [pos], dv, mask=conf)
                offv = offv + plsc.all_reduce_population_count(conf)
            return offv

        offv = lax.fori_loop(0, EBLK // 32, chunk, jnp.zeros((16,), jnp.int32))

        cnt = jnp.max(offv)

        def retry_chunk(j, _):
            valid = (j * 16 + lanes) < offv
            sv = jnp.where(valid, rsrc[pl.ds(j * 16, 16)], 0)
            dv = jnp.where(valid, rdst[pl.ds(j * 16, 16)], 0)
            for k in range(CPW):
                val = plsc.load_gather(xcols[k], [sv])
                cur = plsc.load_gather(aggs[k], [dv])
                m = jnp.where(valid, jnp.maximum(val, cur), NEG)
                plsc.store_scatter(aggs[k], [dv], m, mask=valid)

                def fix_cond(chk):
                    return jnp.any(chk < m)

                def fix_body(chk):
                    plsc.store_scatter(aggs[k], [dv], m, mask=chk < m)
                    return plsc.load_gather(aggs[k], [dv])

                lax.while_loop(fix_cond, fix_body,
                               plsc.load_gather(aggs[k], [dv]))
            return 0

        lax.fori_loop(0, (cnt + 15) // 16, retry_chunk, 0)

    fetch(0, srcb0, dstb0, 0)

    def block_pair(i, _):
        b0 = 2 * i
        b1 = b0 + 1
        fetch(b1, srcb1, dstb1, 2)
        wait(b0, srcb0, dstb0, 0)
        process(srcb0, dstb0)
        fetch(jnp.minimum(b0 + 2, NB - 1), srcb0, dstb0, 0)
        wait(b1, srcb1, dstb1, 2)
        process(srcb1, dstb1)
        return 0

    lax.fori_loop(0, NB // 2, block_pair, 0)
    wait(NB - 1, srcb0, dstb0, 0)
    for k in range(CPW):
        pltpu.sync_copy(aggs[k], aggt_hbm.at[pl.ds((wid * CPW + k) * N, N)])


@functools.partial(jax.jit, static_argnames=())
def _sc_segment_max(xt_flat, src, dst):
    mesh = plsc.VectorSubcoreMesh(core_axis_name="c", subcore_axis_name="s",
                                  num_cores=NC, num_subcores=NS)
    return pl.kernel(
        _sc_segment_max_kernel,
        out_type=jax.ShapeDtypeStruct((D_IN * N,), jnp.float32),
        mesh=mesh,
        compiler_params=pltpu.CompilerParams(needs_layout_passes=False),
        scratch_types=(
            [pltpu.VMEM((N,), jnp.float32) for _ in range(2 * CPW)]
            + [pltpu.VMEM((EBLK,), jnp.int32) for _ in range(6)]
            + [pltpu.SemaphoreType.DMA((4,))]
        ),
    )(xt_flat, src, dst)


def _epilogue_kernel(aggt_ref, xt_ref, wl_ref, wr_ref, b_ref, out_ref):
    aggt = aggt_ref[...]
    aggt = jnp.where(aggt <= NEG, 0.0, aggt)
    outt = (
        lax.dot_general(wl_ref[...], aggt, (((0,), (0,)), ((), ())))
        + lax.dot_general(wr_ref[...], xt_ref[...], (((0,), (0,)), ((), ())))
        + b_ref[...]
    )
    s = outt - jnp.max(outt, axis=0, keepdims=True)
    out_ref[...] = s - jnp.log(jnp.sum(jnp.exp(s), axis=0, keepdims=True))


def _epilogue(aggt, xt, W_l, W_r, b):
    return pl.pallas_call(
        _epilogue_kernel,
        out_shape=jax.ShapeDtypeStruct((D_OUT, N), jnp.float32),
    )(aggt, xt, W_l, W_r, b)


def kernel(x, edge_index, W_l, W_r, b):
    src = edge_index[0].astype(jnp.int32)
    dst = edge_index[1].astype(jnp.int32)
    xt = x.T
    aggt = _sc_segment_max(xt.reshape(-1), src, dst).reshape(D_IN, N)
    outt = _epilogue(aggt, xt, W_l, W_r, b.reshape(D_OUT, 1))
    return outt.T


# speedup vs baseline: 7.2058x; 1.2032x over previous
import functools

import jax
import jax.numpy as jnp
from jax import lax
from jax.experimental import pallas as pl
from jax.experimental.pallas import tpu as pltpu
from jax.experimental.pallas import tpu_sc as plsc

N = 10000
E = 320000
D_IN = 128
D_OUT = 7

NC = 2
NS = 16
NW = NC * NS
CPW = D_IN // NW
EBLK = 8000
NEG = -3.0e38


def _sc_segment_max_kernel(xt_hbm, src_hbm, dst_hbm, aggt_hbm, *scratch):
    xcols = scratch[0:CPW]
    aggs = scratch[CPW:2 * CPW]
    srcb0, dstb0, srcb1, dstb1, rsrc, rdst = scratch[2 * CPW:2 * CPW + 6]
    sems = scratch[2 * CPW + 6]
    wid = lax.axis_index("s") * NC + lax.axis_index("c")
    for k in range(CPW):
        pltpu.sync_copy(xt_hbm.at[pl.ds((wid * CPW + k) * N, N)], xcols[k])

    ninf = jnp.full((16,), NEG, jnp.float32)
    lanes = lax.iota(jnp.int32, 16)

    def init_body(i, _):
        for k in range(CPW):
            aggs[k][pl.ds(i * 16, 16)] = ninf
        return 0

    lax.fori_loop(0, N // 16, init_body, 0)

    NB = E // EBLK

    def fetch(b, srcb, dstb, sp):
        pltpu.async_copy(src_hbm.at[pl.ds(b * EBLK, EBLK)], srcb, sems.at[sp])
        pltpu.async_copy(dst_hbm.at[pl.ds(b * EBLK, EBLK)], dstb,
                         sems.at[sp + 1])

    def wait(b, srcb, dstb, sp):
        pltpu.make_async_copy(src_hbm.at[pl.ds(b * EBLK, EBLK)], srcb,
                              sems.at[sp]).wait()
        pltpu.make_async_copy(dst_hbm.at[pl.ds(b * EBLK, EBLK)], dstb,
                              sems.at[sp + 1]).wait()

    UNR = 4
    NCH = EBLK // (16 * UNR)

    def process(srcb, dstb):
        def load_idx(i):
            svs = [srcb[pl.ds((UNR * i + u) * 16, 16)] for u in range(UNR)]
            dvs = [dstb[pl.ds((UNR * i + u) * 16, 16)] for u in range(UNR)]
            return svs, dvs

        def chunk(i, carry):
            offv, svs, dvs = carry
            scans = [plsc.scan_count(dvs[u]) for u in range(UNR)]
            nsvs, ndvs = load_idx(jnp.minimum(i + 1, NCH - 1))
            vals = [[plsc.load_gather(xcols[k], [svs[u]])
                     for k in range(CPW)] for u in range(UNR)]
            for u in range(UNR):
                sv, dv = svs[u], dvs[u]
                cnt, last = scans[u]
                conf = (cnt != 1) | jnp.logical_not(last)
                curs = [plsc.load_gather(aggs[k], [dv]) for k in range(CPW)]
                for k in range(CPW):
                    plsc.store_scatter(aggs[k], [dv],
                                       jnp.maximum(vals[u][k], curs[k]))
                ranks = plsc.cumsum(conf.astype(jnp.int32))
                pos = offv + ranks - 1
                plsc.store_scatter(rsrc, [pos], sv, mask=conf)
                plsc.store_scatter(rdst, [pos], dv, mask=conf)
                offv = offv + plsc.all_reduce_population_count(conf)
            return offv, nsvs, ndvs

        offv, _, _ = lax.fori_loop(
            0, NCH, chunk,
            (jnp.zeros((16,), jnp.int32),) + load_idx(0))

        cnt = jnp.max(offv)

        def retry_chunk(j, _):
            valid = (j * 16 + lanes) < offv
            sv = jnp.where(valid, rsrc[pl.ds(j * 16, 16)], 0)
            dv = jnp.where(valid, rdst[pl.ds(j * 16, 16)], 0)
            for k in range(CPW):
                val = plsc.load_gather(xcols[k], [sv])
                cur = plsc.load_gather(aggs[k], [dv])
                m = jnp.where(valid, jnp.maximum(val, cur), NEG)
                plsc.store_scatter(aggs[k], [dv], m, mask=valid)

                def fix_cond(chk):
                    return jnp.any(chk < m)

                def fix_body(chk):
                    plsc.store_scatter(aggs[k], [dv], m, mask=chk < m)
                    return plsc.load_gather(aggs[k], [dv])

                lax.while_loop(fix_cond, fix_body,
                               plsc.load_gather(aggs[k], [dv]))
            return 0

        lax.fori_loop(0, (cnt + 15) // 16, retry_chunk, 0)

    fetch(0, srcb0, dstb0, 0)

    def block_pair(i, _):
        b0 = 2 * i
        b1 = b0 + 1
        fetch(b1, srcb1, dstb1, 2)
        wait(b0, srcb0, dstb0, 0)
        process(srcb0, dstb0)
        fetch(jnp.minimum(b0 + 2, NB - 1), srcb0, dstb0, 0)
        wait(b1, srcb1, dstb1, 2)
        process(srcb1, dstb1)
        return 0

    lax.fori_loop(0, NB // 2, block_pair, 0)
    wait(NB - 1, srcb0, dstb0, 0)
    for k in range(CPW):
        pltpu.sync_copy(aggs[k], aggt_hbm.at[pl.ds((wid * CPW + k) * N, N)])


@functools.partial(jax.jit, static_argnames=())
def _sc_segment_max(xt_flat, src, dst):
    mesh = plsc.VectorSubcoreMesh(core_axis_name="c", subcore_axis_name="s",
                                  num_cores=NC, num_subcores=NS)
    return pl.kernel(
        _sc_segment_max_kernel,
        out_type=jax.ShapeDtypeStruct((D_IN * N,), jnp.float32),
        mesh=mesh,
        compiler_params=pltpu.CompilerParams(needs_layout_passes=False),
        scratch_types=(
            [pltpu.VMEM((N,), jnp.float32) for _ in range(2 * CPW)]
            + [pltpu.VMEM((EBLK,), jnp.int32) for _ in range(6)]
            + [pltpu.SemaphoreType.DMA((4,))]
        ),
    )(xt_flat, src, dst)


def _epilogue_kernel(aggt_ref, xt_ref, wl_ref, wr_ref, b_ref, out_ref):
    aggt = aggt_ref[...]
    aggt = jnp.where(aggt <= NEG, 0.0, aggt)
    outt = (
        lax.dot_general(wl_ref[...], aggt, (((0,), (0,)), ((), ())))
        + lax.dot_general(wr_ref[...], xt_ref[...], (((0,), (0,)), ((), ())))
        + b_ref[...]
    )
    s = outt - jnp.max(outt, axis=0, keepdims=True)
    out_ref[...] = s - jnp.log(jnp.sum(jnp.exp(s), axis=0, keepdims=True))


def _epilogue(aggt, xt, W_l, W_r, b):
    return pl.pallas_call(
        _epilogue_kernel,
        out_shape=jax.ShapeDtypeStruct((D_OUT, N), jnp.float32),
    )(aggt, xt, W_l, W_r, b)


def kernel(x, edge_index, W_l, W_r, b):
    src = edge_index[0].astype(jnp.int32)
    dst = edge_index[1].astype(jnp.int32)
    xt = x.T
    aggt = _sc_segment_max(xt.reshape(-1), src, dst).reshape(D_IN, N)
    outt = _epilogue(aggt, xt, W_l, W_r, b.reshape(D_OUT, 1))
    return outt.T
